# baseline (speedup 1.0000x reference)
import functools

import jax
import jax.numpy as jnp
from jax.experimental import pallas as pl
from jax.experimental.pallas import tpu as pltpu

TB = 512
NBUF = 2


def _joint_kernel(lens_ref, f_ref, g_ref, w_ref, bias_ref, out_hbm,
                  ff_vmem, gg_vmem, out_vmem, sems, *, H1, B, NT):
    ti = pl.program_id(0)
    j = pl.program_id(1)

    def out_copy(s_b, s_ti, s):
        return pltpu.make_async_copy(
            out_vmem.at[s],
            out_hbm.at[s_b, pl.ds(s_ti * TB, TB)],
            sems.at[s],
        )

    @pl.when(j == 0)
    def _proj():
        wf = w_ref[:H1, :]
        for b in range(B):
            ff_vmem[b] = (
                jnp.dot(f_ref[:, b, :], wf, preferred_element_type=jnp.float32)
                + bias_ref[0]
            )

        @pl.when(ti == 0)
        def _():
            wg = w_ref[H1:, :]
            for b in range(B):
                gg_vmem[b] = jnp.dot(
                    g_ref[b], wg, preferred_element_type=jnp.float32)

    @pl.when(j > 0)
    def _add():
        b = j - 1
        astep = ti * B + b
        slot = jax.lax.rem(astep, NBUF)

        @pl.when(astep >= NBUF)
        def _():
            prev = astep - NBUF
            out_copy(jax.lax.rem(prev, B), prev // B, slot).wait()

        f_len = lens_ref[0, b]
        g_len = lens_ref[1, b]

        ff = ff_vmem[b]
        gg = gg_vmem[b]
        U, V = gg.shape

        t_ids = ti * TB + jax.lax.broadcasted_iota(jnp.int32, (TB, V), 0)
        u_ids = jax.lax.broadcasted_iota(jnp.int32, (U, V), 0)
        tmask = (t_ids < f_len).astype(jnp.float32)
        umask = (u_ids < g_len).astype(jnp.float32)

        summed = ff[:, None, :] + gg[None, :, :]
        out_vmem[slot] = summed * tmask[:, None, :] * umask[None, :, :]

        out_copy(b, ti, slot).start()

        @pl.when(astep == NT * B - 1)
        def _():
            for k in range(NBUF - 1, -1, -1):
                prev = NT * B - 1 - k
                s = jax.lax.rem(prev, NBUF)
                out_copy(jax.lax.rem(prev, B), prev // B, s).wait()


def kernel(f, f_lens, g, g_lens, W, b):
    T, B, H1 = f.shape
    _, U, H2 = g.shape
    V = W.shape[1]
    NT = T // TB

    lens = jnp.stack([f_lens, g_lens]).astype(jnp.int32)
    bias2d = b.reshape(1, V)

    out = pl.pallas_call(
        functools.partial(_joint_kernel, H1=H1, B=B, NT=NT),
        grid_spec=pltpu.PrefetchScalarGridSpec(
            num_scalar_prefetch=1,
            grid=(NT, 1 + B),
            in_specs=[
                pl.BlockSpec((TB, B, H1), lambda ti, j, lens: (ti, 0, 0)),
                pl.BlockSpec((B, U, H2), lambda ti, j, lens: (0, 0, 0)),
                pl.BlockSpec((H1 + H2, V), lambda ti, j, lens: (0, 0)),
                pl.BlockSpec((1, V), lambda ti, j, lens: (0, 0)),
            ],
            out_specs=pl.BlockSpec(memory_space=pl.ANY),
            scratch_shapes=[
                pltpu.VMEM((B, TB, V), jnp.float32),
                pltpu.VMEM((B, U, V), jnp.float32),
                pltpu.VMEM((NBUF, TB, U, V), jnp.float32),
                pltpu.SemaphoreType.DMA((NBUF,)),
            ],
        ),
        out_shape=jax.ShapeDtypeStruct((B, T, U, V), jnp.float32),
    )(lens, f, g, W, bias2d)
    return (out, f_lens)


# speedup vs baseline: 1.0448x; 1.0448x over previous
import functools

import jax
import jax.numpy as jnp
from jax.experimental import pallas as pl
from jax.experimental.pallas import tpu as pltpu

TB = 256
NBUF = 4


def _joint_kernel(lens_ref, f_ref, g_ref, w_ref, bias_ref, out_hbm,
                  ff_vmem, gg_vmem, out_vmem, sems, *, H1, B, NT):
    ti = pl.program_id(0)
    j = pl.program_id(1)

    def out_copy(s_b, s_ti, s):
        return pltpu.make_async_copy(
            out_vmem.at[s],
            out_hbm.at[s_b, pl.ds(s_ti * TB, TB)],
            sems.at[s],
        )

    @pl.when(j == 0)
    def _proj():
        wf = w_ref[:H1, :]
        for b in range(B):
            ff_vmem[b] = (
                jnp.dot(f_ref[:, b, :], wf, preferred_element_type=jnp.float32)
                + bias_ref[0]
            )

        @pl.when(ti == 0)
        def _():
            wg = w_ref[H1:, :]
            for b in range(B):
                gg_vmem[b] = jnp.dot(
                    g_ref[b], wg, preferred_element_type=jnp.float32)

    @pl.when(j > 0)
    def _add():
        b = j - 1
        astep = ti * B + b
        slot = jax.lax.rem(astep, NBUF)

        @pl.when(astep >= NBUF)
        def _():
            prev = astep - NBUF
            out_copy(jax.lax.rem(prev, B), prev // B, slot).wait()

        f_len = lens_ref[0, b]
        g_len = lens_ref[1, b]

        ff = ff_vmem[b]
        gg = gg_vmem[b]
        U, V = gg.shape

        t_ids = ti * TB + jax.lax.broadcasted_iota(jnp.int32, (TB, V), 0)
        u_ids = jax.lax.broadcasted_iota(jnp.int32, (U, V), 0)
        tmask = (t_ids < f_len).astype(jnp.float32)
        umask = (u_ids < g_len).astype(jnp.float32)

        summed = ff[:, None, :] + gg[None, :, :]
        out_vmem[slot] = summed * tmask[:, None, :] * umask[None, :, :]

        out_copy(b, ti, slot).start()

        @pl.when(astep == NT * B - 1)
        def _():
            for k in range(NBUF - 1, -1, -1):
                prev = NT * B - 1 - k
                s = jax.lax.rem(prev, NBUF)
                out_copy(jax.lax.rem(prev, B), prev // B, s).wait()


def kernel(f, f_lens, g, g_lens, W, b):
    T, B, H1 = f.shape
    _, U, H2 = g.shape
    V = W.shape[1]
    NT = T // TB

    lens = jnp.stack([f_lens, g_lens]).astype(jnp.int32)
    bias2d = b.reshape(1, V)

    out = pl.pallas_call(
        functools.partial(_joint_kernel, H1=H1, B=B, NT=NT),
        grid_spec=pltpu.PrefetchScalarGridSpec(
            num_scalar_prefetch=1,
            grid=(NT, 1 + B),
            in_specs=[
                pl.BlockSpec((TB, B, H1), lambda ti, j, lens: (ti, 0, 0)),
                pl.BlockSpec((B, U, H2), lambda ti, j, lens: (0, 0, 0)),
                pl.BlockSpec((H1 + H2, V), lambda ti, j, lens: (0, 0)),
                pl.BlockSpec((1, V), lambda ti, j, lens: (0, 0)),
            ],
            out_specs=pl.BlockSpec(memory_space=pl.ANY),
            scratch_shapes=[
                pltpu.VMEM((B, TB, V), jnp.float32),
                pltpu.VMEM((B, U, V), jnp.float32),
                pltpu.VMEM((NBUF, TB, U, V), jnp.float32),
                pltpu.SemaphoreType.DMA((NBUF,)),
            ],
        ),
        out_shape=jax.ShapeDtypeStruct((B, T, U, V), jnp.float32),
    )(lens, f, g, W, bias2d)
    return (out, f_lens)


# speedup vs baseline: 1.0470x; 1.0021x over previous
import functools

import jax
import jax.numpy as jnp
from jax.experimental import pallas as pl
from jax.experimental.pallas import tpu as pltpu

TB = 256
NBUF = 5


def _joint_kernel(lens_ref, f_ref, g_ref, w_ref, bias_ref, out_hbm,
                  ff_vmem, gg_vmem, out_vmem, sems, *, H1, B, NT):
    ti = pl.program_id(0)
    j = pl.program_id(1)

    def out_copy(s_b, s_ti, s):
        return pltpu.make_async_copy(
            out_vmem.at[s],
            out_hbm.at[s_b, pl.ds(s_ti * TB, TB)],
            sems.at[s],
        )

    @pl.when(j == 0)
    def _proj():
        wf = w_ref[:H1, :]
        for b in range(B):
            ff_vmem[b] = (
                jnp.dot(f_ref[:, b, :], wf, preferred_element_type=jnp.float32)
                + bias_ref[0]
            )

        @pl.when(ti == 0)
        def _():
            wg = w_ref[H1:, :]
            for b in range(B):
                gg_vmem[b] = jnp.dot(
                    g_ref[b], wg, preferred_element_type=jnp.float32)

    @pl.when(j > 0)
    def _add():
        b = j - 1
        astep = ti * B + b
        slot = jax.lax.rem(astep, NBUF)

        @pl.when(astep >= NBUF)
        def _():
            prev = astep - NBUF
            out_copy(jax.lax.rem(prev, B), prev // B, slot).wait()

        f_len = lens_ref[0, b]
        g_len = lens_ref[1, b]

        ff = ff_vmem[b]
        gg = gg_vmem[b]
        U, V = gg.shape

        t_ids = ti * TB + jax.lax.broadcasted_iota(jnp.int32, (TB, V), 0)
        u_ids = jax.lax.broadcasted_iota(jnp.int32, (U, V), 0)
        tmask = (t_ids < f_len).astype(jnp.float32)
        umask = (u_ids < g_len).astype(jnp.float32)

        summed = ff[:, None, :] + gg[None, :, :]
        out_vmem[slot] = summed * tmask[:, None, :] * umask[None, :, :]

        out_copy(b, ti, slot).start()

        @pl.when(astep == NT * B - 1)
        def _():
            for k in range(NBUF - 1, -1, -1):
                prev = NT * B - 1 - k
                s = jax.lax.rem(prev, NBUF)
                out_copy(jax.lax.rem(prev, B), prev // B, s).wait()


def kernel(f, f_lens, g, g_lens, W, b):
    T, B, H1 = f.shape
    _, U, H2 = g.shape
    V = W.shape[1]
    NT = T // TB

    lens = jnp.stack([f_lens, g_lens]).astype(jnp.int32)
    bias2d = b.reshape(1, V)

    out = pl.pallas_call(
        functools.partial(_joint_kernel, H1=H1, B=B, NT=NT),
        grid_spec=pltpu.PrefetchScalarGridSpec(
            num_scalar_prefetch=1,
            grid=(NT, 1 + B),
            in_specs=[
                pl.BlockSpec((TB, B, H1), lambda ti, j, lens: (ti, 0, 0)),
                pl.BlockSpec((B, U, H2), lambda ti, j, lens: (0, 0, 0)),
                pl.BlockSpec((H1 + H2, V), lambda ti, j, lens: (0, 0)),
                pl.BlockSpec((1, V), lambda ti, j, lens: (0, 0)),
            ],
            out_specs=pl.BlockSpec(memory_space=pl.ANY),
            scratch_shapes=[
                pltpu.VMEM((B, TB, V), jnp.float32),
                pltpu.VMEM((B, U, V), jnp.float32),
                pltpu.VMEM((NBUF, TB, U, V), jnp.float32),
                pltpu.SemaphoreType.DMA((NBUF,)),
            ],
        ),
        out_shape=jax.ShapeDtypeStruct((B, T, U, V), jnp.float32),
    )(lens, f, g, W, bias2d)
    return (out, f_lens)
